# baseline (speedup 1.0000x reference)
import functools

import jax
import jax.numpy as jnp
import numpy as np
from jax import lax
from jax.experimental import pallas as pl
from jax.experimental.pallas import tpu as pltpu
from jax.experimental.pallas import tpu_sc as plsc

B, DIM, H, W = 16, 2, 512, 512
HW = H * W
M = 500
L = 16
NJ = 32
NV = B * M
NR = 8

_LT_THRESH = np.float32(1.0 / 9.0)
_LIN_OFF = np.float32(0.5 / 9.0)

_mesh = plsc.VectorSubcoreMesh(core_axis_name="c", subcore_axis_name="s",
                               num_cores=1)

_DNUMS = lax.GatherDimensionNumbers(
    offset_dims=(), collapsed_slice_dims=(0,), start_index_map=(0,))


def _perm(v, idx):
    return lax.gather(v, idx[:, None], _DNUMS, slice_sizes=(1,),
                      mode=lax.GatherScatterMode.PROMISE_IN_BOUNDS)


@functools.partial(
    pl.kernel,
    mesh=_mesh,
    out_type=(jax.ShapeDtypeStruct((L,), jnp.float32),
              jax.ShapeDtypeStruct((L, 2 * L), jnp.float32)),
    scratch_types=[
        pltpu.VMEM((B * M + 16,), jnp.int32),
        pltpu.VMEM((B * M + 16,), jnp.float32),
        pltpu.VMEM((B * M * DIM + 32,), jnp.float32),
        pltpu.VMEM((NR, 128), jnp.int32),
        pltpu.VMEM((NR, 128), jnp.float32),
        pltpu.VMEM((2 * L,), jnp.float32),
        pltpu.VMEM((L, 2 * L), jnp.float32),
        pltpu.VMEM((L,), jnp.float32),
        pltpu.SemaphoreType.DMA,
    ],
)
def _smooth_reg_loss_sc(feat_hbm, ind_hbm, tgt_hbm, msk_hbm, out_hbm, exch_hbm,
                        ind_all, msk_all, tgt_all, addr_v, pred_v,
                        part_v, all_v, out_v, sem):
    w = lax.axis_index("s")

    pltpu.sync_copy(ind_hbm, ind_all.at[pl.ds(0, NV)])
    pltpu.sync_copy(msk_hbm, msk_all.at[pl.ds(0, NV)])
    pltpu.sync_copy(tgt_hbm, tgt_all.at[pl.ds(0, NV * DIM)])

    lane = lax.broadcasted_iota(jnp.int32, (L,), 0)
    pair_lo = lane >> 1
    pair_hi = pair_lo + 8
    ibase = w * M
    tbase = w * (M * DIM)
    offs_v = w * (DIM * HW) + (lane & 1) * HW

    for j in range(NJ):
        iv = ind_all[pl.ds(ibase + j * L, L)]
        phys = ((iv & -4096)
                + ((iv >> 7) & 3) * 1024
                + ((iv >> 9) & 7) * 128
                + (iv & 127))
        a = _perm(phys, pair_lo) + offs_v
        b = _perm(phys, pair_hi) + offs_v
        if j == NJ - 1:
            a = jnp.where(lane < 8, a, 0)
            b = jnp.where(lane < 0, b, 0)
        r, c = j // 4, (j % 4) * 32
        addr_v[r, pl.ds(c, L)] = a
        addr_v[r, pl.ds(c + L, L)] = b

    copies = [pltpu.async_copy(feat_hbm.at[addr_v.at[r]], pred_v.at[r], sem)
              for r in range(NR)]
    for cpy in copies:
        cpy.wait()

    acc_i = jnp.zeros((L,), jnp.float32)
    accn = jnp.zeros((L,), jnp.float32)

    def smooth(p, t, m):
        mm = jnp.where(t == t, m, 0.0)
        d = jnp.abs(p * mm - t * mm)
        s = d * 3.0
        return jnp.where(d <= _LT_THRESH, 0.5 * (s * s), d - _LIN_OFF)

    for j in range(NJ):
        mv = msk_all[pl.ds(ibase + j * L, L)]
        if j == NJ - 1:
            mv = jnp.where(lane < 4, mv, 0.0)
        accn = accn + mv
        m_a = _perm(mv, pair_lo)
        m_b = _perm(mv, pair_hi)
        t_a = tgt_all[pl.ds(tbase + j * 2 * L, L)]
        t_b = tgt_all[pl.ds(tbase + j * 2 * L + L, L)]
        r, c = j // 4, (j % 4) * 32
        p_a = pred_v[r, pl.ds(c, L)]
        p_b = pred_v[r, pl.ds(c + L, L)]
        la = smooth(p_a, t_a, m_a)
        if j == NJ - 1:
            acc_i = acc_i + jnp.where(lane < 8, la, 0.0)
        else:
            acc_i = acc_i + la + smooth(p_b, t_b, m_b)

    part_v[pl.ds(0, L)] = acc_i
    part_v[pl.ds(L, L)] = accn
    pltpu.sync_copy(part_v, exch_hbm.at[w])
    plsc.subcore_barrier()

    @pl.when(w == 0)
    def _finalize():
        pltpu.sync_copy(exch_hbm, all_v)
        ri = jnp.zeros((L,), jnp.float32)
        rn = jnp.zeros((L,), jnp.float32)
        for i in range(L):
            ri = ri + all_v[i, pl.ds(0, L)]
            rn = rn + all_v[i, pl.ds(L, L)]
        for s in (8, 4, 2):
            ri = ri + _perm(ri, lane ^ s)
        for s in (8, 4, 2, 1):
            rn = rn + _perm(rn, lane ^ s)
        out_v[...] = jnp.where(lane <= 1, ri / (rn + 1e-4), 0.0)
        pltpu.sync_copy(out_v, out_hbm)


def kernel(output, mask, ind, target, sin_loss):
    assert output.shape == (B, DIM, H, W)
    assert ind.shape == (B, M) and target.shape == (B, M, DIM)

    v6 = output.reshape(B, DIM, H // 8, 8, W // 128, 128)
    feat = jnp.transpose(v6, (0, 1, 2, 4, 3, 5)).reshape(B * DIM * HW)

    out16, _ = _smooth_reg_loss_sc(
        feat,
        ind.astype(jnp.int32).reshape(NV),
        target.reshape(NV * DIM),
        mask.astype(jnp.float32).reshape(NV),
    )
    scale = 1.0 - jnp.asarray(sin_loss, jnp.float32)
    return out16[:DIM] * scale


# speedup vs baseline: 1.2280x; 1.2280x over previous
import functools

import jax
import jax.numpy as jnp
import numpy as np
from jax import lax
from jax.experimental import pallas as pl
from jax.experimental.pallas import tpu as pltpu
from jax.experimental.pallas import tpu_sc as plsc

B, DIM, H, W = 16, 2, 512, 512
HW = H * W
M = 500
MP = 512
CH = 128
NCH = MP // CH
L = 16

_LT_THRESH = np.float32(1.0 / 9.0)
_LIN_OFF = np.float32(0.5 / 9.0)

_mesh = plsc.VectorSubcoreMesh(core_axis_name="c", subcore_axis_name="s",
                               num_cores=1)


@functools.partial(
    pl.kernel,
    mesh=_mesh,
    out_type=(jax.ShapeDtypeStruct((L,), jnp.float32),
              jax.ShapeDtypeStruct((L, 3 * L), jnp.float32)),
    scratch_types=[
        pltpu.VMEM((NCH, CH), jnp.int32),
        pltpu.VMEM((NCH, CH), jnp.int32),
        pltpu.VMEM((NCH, CH), jnp.int32),
        pltpu.VMEM((NCH, CH), jnp.float32),
        pltpu.VMEM((NCH, CH), jnp.float32),
        pltpu.VMEM((NCH, CH), jnp.float32),
        pltpu.VMEM((NCH, CH), jnp.float32),
        pltpu.VMEM((NCH, CH), jnp.float32),
        pltpu.VMEM((3 * L,), jnp.float32),
        pltpu.VMEM((L, 3 * L), jnp.float32),
        pltpu.VMEM((L,), jnp.float32),
        pltpu.SemaphoreType.DMA,
    ],
)
def _smooth_reg_loss_sc(feat_hbm, ind_hbm, tgt_hbm, msk_hbm, out_hbm, exch_hbm,
                        idx_v, a0_v, a1_v, p0_v, p1_v, t0_v, t1_v, mk_v,
                        part_v, all_v, out_v, sem):
    w = lax.axis_index("s")

    pltpu.sync_copy(ind_hbm.at[w], idx_v)
    pltpu.sync_copy(tgt_hbm.at[w, 0], t0_v)
    pltpu.sync_copy(tgt_hbm.at[w, 1], t1_v)
    pltpu.sync_copy(msk_hbm.at[w], mk_v)

    base0 = w * (DIM * HW)
    base1 = base0 + HW
    for r in range(NCH):
        for k in range(CH // L):
            sl = pl.ds(k * L, L)
            v = idx_v[r, sl]
            phys = ((v & -4096)
                    + ((v >> 7) & 3) * 1024
                    + ((v >> 9) & 7) * 128
                    + (v & 127))
            a0_v[r, sl] = phys + base0
            a1_v[r, sl] = phys + base1

    copies = []
    for r in range(NCH):
        copies.append(pltpu.async_copy(feat_hbm.at[a0_v.at[r]], p0_v.at[r], sem))
        copies.append(pltpu.async_copy(feat_hbm.at[a1_v.at[r]], p1_v.at[r], sem))
    for c in copies:
        c.wait()

    acc0 = jnp.zeros((L,), jnp.float32)
    acc1 = jnp.zeros((L,), jnp.float32)
    accn = jnp.zeros((L,), jnp.float32)
    for r in range(NCH):
        for k in range(CH // L):
            sl = pl.ds(k * L, L)
            mk = mk_v[r, sl]
            accn = accn + mk

            t0 = t0_v[r, sl]
            m0 = jnp.where(t0 == t0, mk, 0.0)
            d0 = jnp.abs(p0_v[r, sl] * m0 - t0 * m0)
            s0 = d0 * 3.0
            acc0 = acc0 + jnp.where(d0 <= _LT_THRESH, 0.5 * (s0 * s0),
                                    d0 - _LIN_OFF)

            t1 = t1_v[r, sl]
            m1 = jnp.where(t1 == t1, mk, 0.0)
            d1 = jnp.abs(p1_v[r, sl] * m1 - t1 * m1)
            s1 = d1 * 3.0
            acc1 = acc1 + jnp.where(d1 <= _LT_THRESH, 0.5 * (s1 * s1),
                                    d1 - _LIN_OFF)

    part_v[pl.ds(0, L)] = acc0
    part_v[pl.ds(L, L)] = acc1
    part_v[pl.ds(2 * L, L)] = accn
    pltpu.sync_copy(part_v, exch_hbm.at[w])
    plsc.subcore_barrier()

    @pl.when(w == 0)
    def _finalize():
        pltpu.sync_copy(exch_hbm, all_v)
        r0 = jnp.zeros((L,), jnp.float32)
        r1 = jnp.zeros((L,), jnp.float32)
        rn = jnp.zeros((L,), jnp.float32)
        for i in range(L):
            r0 = r0 + all_v[i, pl.ds(0, L)]
            r1 = r1 + all_v[i, pl.ds(L, L)]
            rn = rn + all_v[i, pl.ds(2 * L, L)]
        lane = lax.broadcasted_iota(jnp.int32, (L,), 0)
        dnums = lax.GatherDimensionNumbers(
            offset_dims=(), collapsed_slice_dims=(0,), start_index_map=(0,))

        def lane_sum(v):
            for s in (8, 4, 2, 1):
                perm = lax.gather(
                    v, (lane ^ s)[:, None], dnums, slice_sizes=(1,),
                    mode=lax.GatherScatterMode.PROMISE_IN_BOUNDS)
                v = v + perm
            return v

        denom = lane_sum(rn) + 1e-4
        l0 = lane_sum(r0) / denom
        l1 = lane_sum(r1) / denom
        out_v[...] = jnp.where(lane == 0, l0, jnp.where(lane == 1, l1, 0.0))
        pltpu.sync_copy(out_v, out_hbm)


def kernel(output, mask, ind, target, sin_loss):
    assert output.shape == (B, DIM, H, W)
    assert ind.shape == (B, M) and target.shape == (B, M, DIM)

    v6 = output.reshape(B, DIM, H // 8, 8, W // 128, 128)
    feat = jnp.transpose(v6, (0, 1, 2, 4, 3, 5)).reshape(B * DIM * HW)

    ind_p = jnp.pad(ind.astype(jnp.int32), ((0, 0), (0, MP - M))
                    ).reshape(B, NCH, CH)
    msk_p = jnp.pad(mask.astype(jnp.float32), ((0, 0), (0, MP - M))
                    ).reshape(B, NCH, CH)
    tgt_p = jnp.pad(jnp.transpose(target, (0, 2, 1)),
                    ((0, 0), (0, 0), (0, MP - M))).reshape(B, DIM, NCH, CH)

    out16, _ = _smooth_reg_loss_sc(feat, ind_p, tgt_p, msk_p)
    scale = 1.0 - jnp.asarray(sin_loss, jnp.float32)
    return out16[:DIM] * scale


# speedup vs baseline: 1.2314x; 1.0027x over previous
import functools

import jax
import jax.numpy as jnp
import numpy as np
from jax import lax
from jax.experimental import pallas as pl
from jax.experimental.pallas import tpu as pltpu
from jax.experimental.pallas import tpu_sc as plsc

B, DIM, H, W = 16, 2, 512, 512
HW = H * W
M = 500
MP = 512
CH = 128
NCH = MP // CH
L = 16

_LT_THRESH = np.float32(1.0 / 9.0)
_LIN_OFF = np.float32(0.5 / 9.0)

_mesh = plsc.VectorSubcoreMesh(core_axis_name="c", subcore_axis_name="s",
                               num_cores=1)


@functools.partial(
    pl.kernel,
    mesh=_mesh,
    compiler_params=pltpu.CompilerParams(skip_device_barrier=True),
    out_type=(jax.ShapeDtypeStruct((L,), jnp.float32),
              jax.ShapeDtypeStruct((L, 3 * L), jnp.float32)),
    scratch_types=[
        pltpu.VMEM((NCH, CH), jnp.int32),
        pltpu.VMEM((NCH, CH), jnp.int32),
        pltpu.VMEM((NCH, CH), jnp.int32),
        pltpu.VMEM((NCH, CH), jnp.float32),
        pltpu.VMEM((NCH, CH), jnp.float32),
        pltpu.VMEM((NCH, CH), jnp.float32),
        pltpu.VMEM((NCH, CH), jnp.float32),
        pltpu.VMEM((NCH, CH), jnp.float32),
        pltpu.VMEM((3 * L,), jnp.float32),
        pltpu.VMEM((L, 3 * L), jnp.float32),
        pltpu.VMEM((L,), jnp.float32),
        pltpu.SemaphoreType.DMA,
    ],
)
def _smooth_reg_loss_sc(feat_hbm, ind_hbm, tgt_hbm, msk_hbm, out_hbm, exch_hbm,
                        idx_v, a0_v, a1_v, p0_v, p1_v, t0_v, t1_v, mk_v,
                        part_v, all_v, out_v, sem):
    w = lax.axis_index("s")

    pltpu.sync_copy(ind_hbm.at[w], idx_v)
    pltpu.sync_copy(tgt_hbm.at[w, 0], t0_v)
    pltpu.sync_copy(tgt_hbm.at[w, 1], t1_v)
    pltpu.sync_copy(msk_hbm.at[w], mk_v)

    base0 = w * (DIM * HW)
    base1 = base0 + HW
    for r in range(NCH):
        for k in range(CH // L):
            sl = pl.ds(k * L, L)
            v = idx_v[r, sl]
            phys = ((v & -4096)
                    + ((v >> 7) & 3) * 1024
                    + ((v >> 9) & 7) * 128
                    + (v & 127))
            a0_v[r, sl] = phys + base0
            a1_v[r, sl] = phys + base1

    copies = []
    for r in range(NCH):
        copies.append(pltpu.async_copy(feat_hbm.at[a0_v.at[r]], p0_v.at[r], sem))
        copies.append(pltpu.async_copy(feat_hbm.at[a1_v.at[r]], p1_v.at[r], sem))
    for c in copies:
        c.wait()

    acc0 = jnp.zeros((L,), jnp.float32)
    acc1 = jnp.zeros((L,), jnp.float32)
    accn = jnp.zeros((L,), jnp.float32)
    for r in range(NCH):
        for k in range(CH // L):
            sl = pl.ds(k * L, L)
            mk = mk_v[r, sl]
            accn = accn + mk

            t0 = t0_v[r, sl]
            m0 = jnp.where(t0 == t0, mk, 0.0)
            d0 = jnp.abs(p0_v[r, sl] * m0 - t0 * m0)
            s0 = d0 * 3.0
            acc0 = acc0 + jnp.where(d0 <= _LT_THRESH, 0.5 * (s0 * s0),
                                    d0 - _LIN_OFF)

            t1 = t1_v[r, sl]
            m1 = jnp.where(t1 == t1, mk, 0.0)
            d1 = jnp.abs(p1_v[r, sl] * m1 - t1 * m1)
            s1 = d1 * 3.0
            acc1 = acc1 + jnp.where(d1 <= _LT_THRESH, 0.5 * (s1 * s1),
                                    d1 - _LIN_OFF)

    part_v[pl.ds(0, L)] = acc0
    part_v[pl.ds(L, L)] = acc1
    part_v[pl.ds(2 * L, L)] = accn
    pltpu.sync_copy(part_v, exch_hbm.at[w])
    plsc.subcore_barrier()

    @pl.when(w == 0)
    def _finalize():
        pltpu.sync_copy(exch_hbm, all_v)
        r0 = jnp.zeros((L,), jnp.float32)
        r1 = jnp.zeros((L,), jnp.float32)
        rn = jnp.zeros((L,), jnp.float32)
        for i in range(L):
            r0 = r0 + all_v[i, pl.ds(0, L)]
            r1 = r1 + all_v[i, pl.ds(L, L)]
            rn = rn + all_v[i, pl.ds(2 * L, L)]
        lane = lax.broadcasted_iota(jnp.int32, (L,), 0)
        dnums = lax.GatherDimensionNumbers(
            offset_dims=(), collapsed_slice_dims=(0,), start_index_map=(0,))

        def lane_sum(v):
            for s in (8, 4, 2, 1):
                perm = lax.gather(
                    v, (lane ^ s)[:, None], dnums, slice_sizes=(1,),
                    mode=lax.GatherScatterMode.PROMISE_IN_BOUNDS)
                v = v + perm
            return v

        denom = lane_sum(rn) + 1e-4
        l0 = lane_sum(r0) / denom
        l1 = lane_sum(r1) / denom
        out_v[...] = jnp.where(lane == 0, l0, jnp.where(lane == 1, l1, 0.0))
        pltpu.sync_copy(out_v, out_hbm)


def kernel(output, mask, ind, target, sin_loss):
    assert output.shape == (B, DIM, H, W)
    assert ind.shape == (B, M) and target.shape == (B, M, DIM)

    v6 = output.reshape(B, DIM, H // 8, 8, W // 128, 128)
    feat = jnp.transpose(v6, (0, 1, 2, 4, 3, 5)).reshape(B * DIM * HW)

    ind_p = jnp.pad(ind.astype(jnp.int32), ((0, 0), (0, MP - M))
                    ).reshape(B, NCH, CH)
    msk_p = jnp.pad(mask.astype(jnp.float32), ((0, 0), (0, MP - M))
                    ).reshape(B, NCH, CH)
    tgt_p = jnp.pad(jnp.transpose(target, (0, 2, 1)),
                    ((0, 0), (0, 0), (0, MP - M))).reshape(B, DIM, NCH, CH)

    out16, _ = _smooth_reg_loss_sc(feat, ind_p, tgt_p, msk_p)
    scale = 1.0 - jnp.asarray(sin_loss, jnp.float32)
    return out16[:DIM] * scale
